# baseline (speedup 1.0000x reference)
import functools

import jax
import jax.numpy as jnp
from jax.experimental import pallas as pl

_BM = 800


def _fused_heads_kernel(x_ref, wc_ref, wb_ref, bc_ref, bb_ref,
                        scores_ref, deltas_ref):
    x = x_ref[...]
    dn = (((1,), (1,)), ((), ()))
    scores_ref[...] = jax.lax.dot_general(
        x, wc_ref[...], dn, preferred_element_type=jnp.float32) + bc_ref[...]
    deltas_ref[...] = jax.lax.dot_general(
        x, wb_ref[...], dn, preferred_element_type=jnp.float32) + bb_ref[...]


@functools.partial(jax.jit, static_argnames=("interpret",))
def _run(x, W_cls, b_cls, W_box, b_box, interpret=False):
    n, d = x.shape
    c1 = W_cls.shape[0]
    c4 = W_box.shape[0]
    grid = (n // _BM,)
    scores, deltas = pl.pallas_call(
        _fused_heads_kernel,
        grid=grid,
        in_specs=[
            pl.BlockSpec((_BM, d), lambda i: (i, 0)),
            pl.BlockSpec((c1, d), lambda i: (0, 0)),
            pl.BlockSpec((c4, d), lambda i: (0, 0)),
            pl.BlockSpec((1, c1), lambda i: (0, 0)),
            pl.BlockSpec((1, c4), lambda i: (0, 0)),
        ],
        out_specs=[
            pl.BlockSpec((_BM, c1), lambda i: (i, 0)),
            pl.BlockSpec((_BM, c4), lambda i: (i, 0)),
        ],
        out_shape=[
            jax.ShapeDtypeStruct((n, c1), jnp.float32),
            jax.ShapeDtypeStruct((n, c4), jnp.float32),
        ],
        interpret=interpret,
    )(x, W_cls, W_box, b_cls.reshape(1, c1), b_box.reshape(1, c4))
    return scores, deltas


def kernel(x, W_cls, b_cls, W_box, b_box):
    if x.ndim > 2:
        x = x.reshape(x.shape[0], -1)
    return _run(x, W_cls, b_cls, W_box, b_box)
